# baseline (speedup 1.0000x reference)
import functools

import jax
import jax.numpy as jnp
from jax import lax
from jax.experimental import pallas as pl
from jax.experimental.pallas import tpu as pltpu
from jax.experimental.pallas import tpu_sc as plsc

N = 10000
NPAD = 10240
E = 320000
D = 128
K1 = 16
K2 = 8
NC = 2
NS = 16
NW = NC * NS
CH = 80
BB = 1280
RPT = NPAD // NS
CPWS = (64, 61)
_SC_MESH = plsc.VectorSubcoreMesh(core_axis_name="c", subcore_axis_name="s")
_F32 = jnp.float32
_PREC = None


def _half_sizes():
    sizes = [cpw * CH * NW for cpw in CPWS]
    assert sum(sizes) == E and all(sz % BB == 0 for sz in sizes)
    return sizes


HS = _half_sizes()
HOFF = (0, HS[0])


DP = D // 2


def _make_gather(cpw):
    eh = cpw * CH * NW

    @functools.partial(
        pl.kernel,
        mesh=_SC_MESH,
        compiler_params=pltpu.CompilerParams(use_tc_tiling_on_sc=False),
        out_type=(
            jax.ShapeDtypeStruct((eh, DP), jnp.int32),
            jax.ShapeDtypeStruct((eh, DP), jnp.int32),
        ),
        scratch_types=[
            pltpu.VMEM((cpw, CH), jnp.int32),
            pltpu.VMEM((cpw, CH), jnp.int32),
            pltpu.VMEM((2, CH, DP), jnp.int32),
            pltpu.VMEM((2, CH, DP), jnp.int32),
            pltpu.SemaphoreType.DMA((2,)),
            pltpu.SemaphoreType.DMA((2,)),
        ],
    )
    def gather(src3d, tgt3d, table, ni_out, nj_out,
               sidx_v, tidx_v, srow2, trow2, sem_g, sem_s):
        c = lax.axis_index("c")
        s = lax.axis_index("s")
        wid = s * NC + c
        row0 = wid * cpw
        pltpu.sync_copy(src3d.at[wid], sidx_v)
        pltpu.sync_copy(tgt3d.at[wid], tidx_v)

        def g_desc(j, slot):
            return (pltpu.make_async_copy(table.at[sidx_v.at[j]],
                                          srow2.at[slot], sem_g.at[slot]),
                    pltpu.make_async_copy(table.at[tidx_v.at[j]],
                                          trow2.at[slot], sem_g.at[slot]))

        def s_desc(j, slot):
            o = (row0 + j) * CH
            return (pltpu.make_async_copy(srow2.at[slot],
                                          ni_out.at[pl.ds(o, CH), :],
                                          sem_s.at[slot]),
                    pltpu.make_async_copy(trow2.at[slot],
                                          nj_out.at[pl.ds(o, CH), :],
                                          sem_s.at[slot]))

        for d in g_desc(0, 0):
            d.start()

        def body(j, carry):
            slot = lax.rem(j, 2)
            nslot = 1 - slot

            @pl.when(j + 1 < cpw)
            def _():
                @pl.when(j >= 1)
                def _():
                    for d in s_desc(j - 1, nslot):
                        d.wait()
                for d in g_desc(j + 1, nslot):
                    d.start()

            for d in g_desc(j, slot):
                d.wait()
            for d in s_desc(j, slot):
                d.start()
            return carry

        lax.fori_loop(0, cpw, body, 0)
        for d in s_desc(cpw - 2, (cpw - 2) % 2):
            d.wait()
        for d in s_desc(cpw - 1, (cpw - 1) % 2):
            d.wait()

    return gather


_GATHERS = tuple(_make_gather(cpw) for cpw in CPWS)


def _unpack(p):
    even = lax.bitcast_convert_type(lax.shift_left(p, 16), _F32)
    odd = lax.bitcast_convert_type(
        jnp.bitwise_and(p, jnp.int32(-65536)), _F32)
    return jnp.concatenate([even, odd], axis=1)


def _compute_x(nip, njp, r, wag, wbg, wcg, wam, wbm, wcm):
    ni = _unpack(nip)
    nj = _unpack(njp)
    delta = (ni - nj) / r
    xg = (jnp.dot(ni, wag, precision=_PREC)
          + jnp.dot(nj, wbg, precision=_PREC)
          + jnp.dot(delta, wcg, precision=_PREC))
    xm = (jnp.dot(ni, wam, precision=_PREC)
          + jnp.dot(nj, wbm, precision=_PREC)
          + jnp.dot(delta, wcm, precision=_PREC))
    return xg, xm


def _passA_body(ni_ref, nj_ref, r_ref, pw_ref,
                wag, wbg, wcg, wam, wbm, wcm, w2gt,
                sg, qg, sm, qm, sy, qy):
    i = pl.program_id(0)
    xg, xm = _compute_x(ni_ref[...], nj_ref[...], r_ref[...],
                        wag[...], wbg[...], wcg[...],
                        wam[...], wbm[...], wcm[...])
    y = jnp.dot(pw_ref[...], w2gt[...], precision=_PREC)
    bs_g = jnp.sum(xg, axis=0, keepdims=True)
    bq_g = jnp.sum(xg * xg, axis=0, keepdims=True)
    bs_m = jnp.sum(xm, axis=0, keepdims=True)
    bq_m = jnp.sum(xm * xm, axis=0, keepdims=True)
    bs_y = jnp.sum(y, axis=0, keepdims=True)
    bq_y = jnp.sum(y * y, axis=0, keepdims=True)

    @pl.when(i == 0)
    def _():
        sg[...] = bs_g
        qg[...] = bq_g
        sm[...] = bs_m
        qm[...] = bq_m
        sy[...] = bs_y
        qy[...] = bq_y

    @pl.when(i != 0)
    def _():
        sg[...] += bs_g
        qg[...] += bq_g
        sm[...] += bs_m
        qm[...] += bq_m
        sy[...] += bs_y
        qy[...] += bq_y


def _run_passA(ni, nj, r2d, pw, wag, wbg, wcg, wam, wbm, wcm, w2gt):
    nblk = ni.shape[0] // BB
    blk = lambda i: (i, 0)
    cst = lambda i: (0, 0)
    return pl.pallas_call(
        _passA_body,
        grid=(nblk,),
        in_specs=[
            pl.BlockSpec((BB, DP), blk),
            pl.BlockSpec((BB, DP), blk),
            pl.BlockSpec((BB, 1), blk),
            pl.BlockSpec((BB, K2), blk),
            pl.BlockSpec((D, D), cst),
            pl.BlockSpec((D, D), cst),
            pl.BlockSpec((D, D), cst),
            pl.BlockSpec((D, D), cst),
            pl.BlockSpec((D, D), cst),
            pl.BlockSpec((D, D), cst),
            pl.BlockSpec((K2, K2), cst),
        ],
        out_specs=[
            pl.BlockSpec((1, D), cst),
            pl.BlockSpec((1, D), cst),
            pl.BlockSpec((1, D), cst),
            pl.BlockSpec((1, D), cst),
            pl.BlockSpec((1, K2), cst),
            pl.BlockSpec((1, K2), cst),
        ],
        out_shape=[
            jax.ShapeDtypeStruct((1, D), _F32),
            jax.ShapeDtypeStruct((1, D), _F32),
            jax.ShapeDtypeStruct((1, D), _F32),
            jax.ShapeDtypeStruct((1, D), _F32),
            jax.ShapeDtypeStruct((1, K2), _F32),
            jax.ShapeDtypeStruct((1, K2), _F32),
        ],
    )(ni, nj, r2d, pw, wag, wbg, wcg, wam, wbm, wcm, w2gt)


def _passB_body(ni_ref, nj_ref, r_ref, cs_ref, pw_ref,
                wag, wbg, wcg, wam, wbm, wcm, cutf,
                scg, shg, scm, shm, w1t, b1, w2t, b2, w2gt, sc2, sh2,
                z_ref):
    xg, xm = _compute_x(ni_ref[...], nj_ref[...], r_ref[...],
                        wag[...], wbg[...], wcg[...],
                        wam[...], wbm[...], wcm[...])
    eg = xg * scg[...] + shg[...]
    em = xm * scm[...] + shm[...]
    z1 = jnp.dot(cs_ref[...], w1t[...], precision=_PREC) + b1[...]
    pw = pw_ref[...]
    y = jnp.dot(pw, w2gt[...], precision=_PREC)
    gate = y * sc2[...] + sh2[...]
    z2 = jnp.dot(pw * gate, w2t[...], precision=_PREC) + b2[...]
    mask = (r_ref[...] < cutf[...]).astype(_F32)
    z_ref[...] = eg * em * (z1 + z2) * mask


def _run_passB(ni, nj, r2d, cs, pw, wag, wbg, wcg, wam, wbm, wcm,
               cutf, scg, shg, scm, shm, w1t, b1, w2t, b2, w2gt, sc2, sh2):
    nblk = ni.shape[0] // BB
    blk = lambda i: (i, 0)
    cst = lambda i: (0, 0)
    return pl.pallas_call(
        _passB_body,
        grid=(nblk,),
        in_specs=[
            pl.BlockSpec((BB, DP), blk),
            pl.BlockSpec((BB, DP), blk),
            pl.BlockSpec((BB, 1), blk),
            pl.BlockSpec((BB, K1), blk),
            pl.BlockSpec((BB, K2), blk),
            pl.BlockSpec((D, D), cst),
            pl.BlockSpec((D, D), cst),
            pl.BlockSpec((D, D), cst),
            pl.BlockSpec((D, D), cst),
            pl.BlockSpec((D, D), cst),
            pl.BlockSpec((D, D), cst),
            pl.BlockSpec((1, 1), cst),
            pl.BlockSpec((1, D), cst),
            pl.BlockSpec((1, D), cst),
            pl.BlockSpec((1, D), cst),
            pl.BlockSpec((1, D), cst),
            pl.BlockSpec((K1, D), cst),
            pl.BlockSpec((1, D), cst),
            pl.BlockSpec((K2, D), cst),
            pl.BlockSpec((1, D), cst),
            pl.BlockSpec((K2, K2), cst),
            pl.BlockSpec((1, K2), cst),
            pl.BlockSpec((1, K2), cst),
        ],
        out_specs=pl.BlockSpec((BB, D), blk),
        out_shape=jax.ShapeDtypeStruct((ni.shape[0], D), _F32),
    )(ni, nj, r2d, cs, pw, wag, wbg, wcg, wam, wbm, wcm,
      cutf, scg, shg, scm, shm, w1t, b1, w2t, b2, w2gt, sc2, sh2)


def _make_scatter(cpw):
    eh = cpw * CH * NW

    @functools.partial(
        pl.kernel,
        mesh=_SC_MESH,
        out_type=jax.ShapeDtypeStruct((NC, NPAD, D), _F32),
        scratch_types=[
            pltpu.VMEM((cpw, CH), jnp.int32),
            pltpu.VMEM((2, CH, D), _F32),
            pltpu.SemaphoreType.DMA((2,)),
            pltpu.SemaphoreType.DMA((2,)),
            pltpu.VMEM_SHARED((NPAD, D), _F32),
        ],
    )
    def scatter(src3d, z_hbm, zeros_hbm, part_out,
                sidx_v, zrow2, sem_l, sem_a, acc):
        c = lax.axis_index("c")
        s = lax.axis_index("s")
        wid = s * NC + c
        pltpu.sync_copy(zeros_hbm.at[pl.ds(s * RPT, RPT), :],
                        acc.at[pl.ds(s * RPT, RPT), :])
        plsc.subcore_barrier()
        pltpu.sync_copy(src3d.at[wid], sidx_v)

        def l_desc(j, slot):
            o = (wid * cpw + j) * CH
            return pltpu.make_async_copy(z_hbm.at[pl.ds(o, CH), :],
                                         zrow2.at[slot], sem_l.at[slot])

        def a_desc(j, slot):
            return pltpu.make_async_copy(zrow2.at[slot],
                                         acc.at[sidx_v.at[j]],
                                         sem_a.at[slot])

        l_desc(0, 0).start()

        def body(j, carry):
            slot = lax.rem(j, 2)
            nslot = 1 - slot

            @pl.when(j + 1 < cpw)
            def _():
                @pl.when(j >= 1)
                def _():
                    a_desc(j - 1, nslot).wait()
                l_desc(j + 1, nslot).start()

            l_desc(j, slot).wait()
            a_desc(j, slot).start(add=True)
            return carry

        lax.fori_loop(0, cpw, body, 0)
        a_desc(cpw - 2, (cpw - 2) % 2).wait()
        a_desc(cpw - 1, (cpw - 1) % 2).wait()
        plsc.subcore_barrier()
        pltpu.sync_copy(acc.at[pl.ds(s * RPT, RPT), :],
                        part_out.at[c, pl.ds(s * RPT, RPT), :])

    return scatter


_SCATTERS = tuple(_make_scatter(cpw) for cpw in CPWS)


def _combine_body(inp_ref, a_ref, b_ref, c_ref, d_ref, out_ref):
    out_ref[...] = (inp_ref[...] + a_ref[...] + b_ref[...]
                    + c_ref[...] + d_ref[...])


def _run_combine(inp, pa, pb, pc, pd):
    blk = lambda i: (i, 0)
    return pl.pallas_call(
        _combine_body,
        grid=(5,),
        in_specs=[pl.BlockSpec((2000, D), blk)] * 5,
        out_specs=pl.BlockSpec((2000, D), blk),
        out_shape=jax.ShapeDtypeStruct((N, D), _F32),
    )(inp, pa, pb, pc, pd)


def kernel(input, edge_sources, edge_targets, rij, combine_sets, plane_wave,
           cutoff, W_gate, b_gate, g_gate, be_gate, W_mlp, b_mlp, g_mlp,
           be_mlp, W1, b1, W2, b2, W2g, b2g, g2, be2):
    f32 = _F32
    esrc = edge_sources.astype(jnp.int32)
    etgt = edge_targets.astype(jnp.int32)

    def half(x, h):
        return x[HOFF[h]:HOFF[h] + HS[h]]

    src3d = [half(esrc, h).reshape(NW, CPWS[h], CH) for h in range(2)]
    tgt3d = [half(etgt, h).reshape(NW, CPWS[h], CH) for h in range(2)]
    r2d = [half(rij, h).reshape(HS[h], 1) for h in range(2)]
    csh = [half(combine_sets, h) for h in range(2)]
    pwh = [half(plane_wave, h) for h in range(2)]

    table_p = lax.bitcast_convert_type(
        input.astype(jnp.bfloat16).reshape(N, DP, 2), jnp.int32)
    pairs = [_GATHERS[h](src3d[h], tgt3d[h], table_p) for h in range(2)]

    perm = jnp.arange(D).reshape(DP, 2).T.reshape(D)
    wag = W_gate[:, :D].T[perm]
    wbg = W_gate[:, D:2 * D].T[perm]
    wcg = W_gate[:, 2 * D:].T[perm]
    wam = W_mlp[:, :D].T[perm]
    wbm = W_mlp[:, D:2 * D].T[perm]
    wcm = W_mlp[:, 2 * D:].T[perm]
    w2gt = W2g.T

    stats = [_run_passA(pairs[h][0], pairs[h][1], r2d[h], pwh[h],
                        wag, wbg, wcg, wam, wbm, wcm, w2gt)
             for h in range(2)]
    sg, qg, sm, qm, sy, qy = [a + b for a, b in zip(*stats)]

    eps = 1e-5
    inv_e = 1.0 / E

    def scale_shift(s_, q_, g_, be_):
        mean = s_ * inv_e
        var = q_ * inv_e - mean * mean
        inv = g_.reshape(1, -1) / jnp.sqrt(var + eps)
        return inv, be_.reshape(1, -1) - mean * inv

    scg, shg = scale_shift(sg, qg, g_gate, be_gate)
    scm, shm = scale_shift(sm, qm, g_mlp, be_mlp)
    sc2, sh2 = scale_shift(sy, qy, g2, be2)

    cutf = jnp.full((1, 1), cutoff, f32)
    zeros = jnp.zeros((NPAD, D), f32)
    parts = []
    for h in range(2):
        z = _run_passB(pairs[h][0], pairs[h][1], r2d[h], csh[h], pwh[h],
                       wag, wbg, wcg, wam, wbm, wcm, cutf,
                       scg, shg, scm, shm,
                       W1.T, b1.reshape(1, D), W2.T, b2.reshape(1, D),
                       w2gt, sc2, sh2)
        parts.append(_SCATTERS[h](src3d[h], z, zeros))

    return _run_combine(input, parts[0][0, :N], parts[0][1, :N],
                        parts[1][0, :N], parts[1][1, :N])


# speedup vs baseline: 1.1637x; 1.1637x over previous
import functools

import jax
import jax.numpy as jnp
from jax import lax
from jax.experimental import pallas as pl
from jax.experimental.pallas import tpu as pltpu
from jax.experimental.pallas import tpu_sc as plsc

N = 10000
NPAD = 10240
E = 320000
D = 128
K1 = 16
K2 = 8
NC = 2
NS = 16
NW = NC * NS
CH = 80
BB = 1280
RPT = NPAD // NS
CPWS = (64, 61)
_SC_MESH = plsc.VectorSubcoreMesh(core_axis_name="c", subcore_axis_name="s")
_F32 = jnp.float32
_PREC = None


def _half_sizes():
    sizes = [cpw * CH * NW for cpw in CPWS]
    assert sum(sizes) == E and all(sz % BB == 0 for sz in sizes)
    return sizes


HS = _half_sizes()
HOFF = (0, HS[0])


def _make_gather(cpw):
    eh = cpw * CH * NW

    @functools.partial(
        pl.kernel,
        mesh=_SC_MESH,
        out_type=(
            jax.ShapeDtypeStruct((eh, D), _F32),
            jax.ShapeDtypeStruct((eh, D), _F32),
        ),
        scratch_types=[
            pltpu.VMEM((cpw, CH), jnp.int32),
            pltpu.VMEM((cpw, CH), jnp.int32),
            pltpu.VMEM((2, CH, D), _F32),
            pltpu.VMEM((2, CH, D), _F32),
            pltpu.SemaphoreType.DMA((2,)),
            pltpu.SemaphoreType.DMA((2,)),
        ],
    )
    def gather(src3d, tgt3d, table, ni_out, nj_out,
               sidx_v, tidx_v, srow2, trow2, sem_g, sem_s):
        c = lax.axis_index("c")
        s = lax.axis_index("s")
        wid = s * NC + c
        row0 = wid * cpw
        pltpu.sync_copy(src3d.at[wid], sidx_v)
        pltpu.sync_copy(tgt3d.at[wid], tidx_v)

        def g_desc(j, slot):
            return (pltpu.make_async_copy(table.at[sidx_v.at[j]],
                                          srow2.at[slot], sem_g.at[slot]),
                    pltpu.make_async_copy(table.at[tidx_v.at[j]],
                                          trow2.at[slot], sem_g.at[slot]))

        def s_desc(j, slot):
            o = (row0 + j) * CH
            return (pltpu.make_async_copy(srow2.at[slot],
                                          ni_out.at[pl.ds(o, CH), :],
                                          sem_s.at[slot]),
                    pltpu.make_async_copy(trow2.at[slot],
                                          nj_out.at[pl.ds(o, CH), :],
                                          sem_s.at[slot]))

        for d in g_desc(0, 0):
            d.start()

        def body(j, carry):
            slot = lax.rem(j, 2)
            nslot = 1 - slot

            @pl.when(j + 1 < cpw)
            def _():
                @pl.when(j >= 1)
                def _():
                    for d in s_desc(j - 1, nslot):
                        d.wait()
                for d in g_desc(j + 1, nslot):
                    d.start()

            for d in g_desc(j, slot):
                d.wait()
            for d in s_desc(j, slot):
                d.start()
            return carry

        lax.fori_loop(0, cpw, body, 0)
        for d in s_desc(cpw - 2, (cpw - 2) % 2):
            d.wait()
        for d in s_desc(cpw - 1, (cpw - 1) % 2):
            d.wait()

    return gather


_GATHERS = tuple(_make_gather(cpw) for cpw in CPWS)


def _compute_x(ni, nj, r, wag, wbg, wcg, wam, wbm, wcm):
    delta = (ni - nj) / r
    xg = (jnp.dot(ni, wag, precision=_PREC)
          + jnp.dot(nj, wbg, precision=_PREC)
          + jnp.dot(delta, wcg, precision=_PREC))
    xm = (jnp.dot(ni, wam, precision=_PREC)
          + jnp.dot(nj, wbm, precision=_PREC)
          + jnp.dot(delta, wcm, precision=_PREC))
    return xg, xm


def _passA_body(ni_ref, nj_ref, r_ref, pw_ref,
                wag, wbg, wcg, wam, wbm, wcm, w2gt,
                sg, qg, sm, qm, sy, qy):
    i = pl.program_id(0)
    xg, xm = _compute_x(ni_ref[...], nj_ref[...], r_ref[...],
                        wag[...], wbg[...], wcg[...],
                        wam[...], wbm[...], wcm[...])
    y = jnp.dot(pw_ref[...], w2gt[...], precision=_PREC)
    bs_g = jnp.sum(xg, axis=0, keepdims=True)
    bq_g = jnp.sum(xg * xg, axis=0, keepdims=True)
    bs_m = jnp.sum(xm, axis=0, keepdims=True)
    bq_m = jnp.sum(xm * xm, axis=0, keepdims=True)
    bs_y = jnp.sum(y, axis=0, keepdims=True)
    bq_y = jnp.sum(y * y, axis=0, keepdims=True)

    @pl.when(i == 0)
    def _():
        sg[...] = bs_g
        qg[...] = bq_g
        sm[...] = bs_m
        qm[...] = bq_m
        sy[...] = bs_y
        qy[...] = bq_y

    @pl.when(i != 0)
    def _():
        sg[...] += bs_g
        qg[...] += bq_g
        sm[...] += bs_m
        qm[...] += bq_m
        sy[...] += bs_y
        qy[...] += bq_y


def _run_passA(ni, nj, r2d, pw, wag, wbg, wcg, wam, wbm, wcm, w2gt):
    nblk = ni.shape[0] // BB
    blk = lambda i: (i, 0)
    cst = lambda i: (0, 0)
    return pl.pallas_call(
        _passA_body,
        grid=(nblk,),
        in_specs=[
            pl.BlockSpec((BB, D), blk),
            pl.BlockSpec((BB, D), blk),
            pl.BlockSpec((BB, 1), blk),
            pl.BlockSpec((BB, K2), blk),
            pl.BlockSpec((D, D), cst),
            pl.BlockSpec((D, D), cst),
            pl.BlockSpec((D, D), cst),
            pl.BlockSpec((D, D), cst),
            pl.BlockSpec((D, D), cst),
            pl.BlockSpec((D, D), cst),
            pl.BlockSpec((K2, K2), cst),
        ],
        out_specs=[
            pl.BlockSpec((1, D), cst),
            pl.BlockSpec((1, D), cst),
            pl.BlockSpec((1, D), cst),
            pl.BlockSpec((1, D), cst),
            pl.BlockSpec((1, K2), cst),
            pl.BlockSpec((1, K2), cst),
        ],
        out_shape=[
            jax.ShapeDtypeStruct((1, D), _F32),
            jax.ShapeDtypeStruct((1, D), _F32),
            jax.ShapeDtypeStruct((1, D), _F32),
            jax.ShapeDtypeStruct((1, D), _F32),
            jax.ShapeDtypeStruct((1, K2), _F32),
            jax.ShapeDtypeStruct((1, K2), _F32),
        ],
    )(ni, nj, r2d, pw, wag, wbg, wcg, wam, wbm, wcm, w2gt)


def _passB_body(ni_ref, nj_ref, r_ref, cs_ref, pw_ref,
                wag, wbg, wcg, wam, wbm, wcm, cutf,
                scg, shg, scm, shm, w1t, b1, w2t, b2, w2gt, sc2, sh2,
                z_ref):
    xg, xm = _compute_x(ni_ref[...], nj_ref[...], r_ref[...],
                        wag[...], wbg[...], wcg[...],
                        wam[...], wbm[...], wcm[...])
    eg = xg * scg[...] + shg[...]
    em = xm * scm[...] + shm[...]
    z1 = jnp.dot(cs_ref[...], w1t[...], precision=_PREC) + b1[...]
    pw = pw_ref[...]
    y = jnp.dot(pw, w2gt[...], precision=_PREC)
    gate = y * sc2[...] + sh2[...]
    z2 = jnp.dot(pw * gate, w2t[...], precision=_PREC) + b2[...]
    mask = (r_ref[...] < cutf[...]).astype(_F32)
    z_ref[...] = eg * em * (z1 + z2) * mask


def _run_passB(ni, nj, r2d, cs, pw, wag, wbg, wcg, wam, wbm, wcm,
               cutf, scg, shg, scm, shm, w1t, b1, w2t, b2, w2gt, sc2, sh2):
    nblk = ni.shape[0] // BB
    blk = lambda i: (i, 0)
    cst = lambda i: (0, 0)
    return pl.pallas_call(
        _passB_body,
        grid=(nblk,),
        in_specs=[
            pl.BlockSpec((BB, D), blk),
            pl.BlockSpec((BB, D), blk),
            pl.BlockSpec((BB, 1), blk),
            pl.BlockSpec((BB, K1), blk),
            pl.BlockSpec((BB, K2), blk),
            pl.BlockSpec((D, D), cst),
            pl.BlockSpec((D, D), cst),
            pl.BlockSpec((D, D), cst),
            pl.BlockSpec((D, D), cst),
            pl.BlockSpec((D, D), cst),
            pl.BlockSpec((D, D), cst),
            pl.BlockSpec((1, 1), cst),
            pl.BlockSpec((1, D), cst),
            pl.BlockSpec((1, D), cst),
            pl.BlockSpec((1, D), cst),
            pl.BlockSpec((1, D), cst),
            pl.BlockSpec((K1, D), cst),
            pl.BlockSpec((1, D), cst),
            pl.BlockSpec((K2, D), cst),
            pl.BlockSpec((1, D), cst),
            pl.BlockSpec((K2, K2), cst),
            pl.BlockSpec((1, K2), cst),
            pl.BlockSpec((1, K2), cst),
        ],
        out_specs=pl.BlockSpec((BB, D), blk),
        out_shape=jax.ShapeDtypeStruct((ni.shape[0], D), _F32),
    )(ni, nj, r2d, cs, pw, wag, wbg, wcg, wam, wbm, wcm,
      cutf, scg, shg, scm, shm, w1t, b1, w2t, b2, w2gt, sc2, sh2)


def _make_scatter(cpw):
    eh = cpw * CH * NW

    @functools.partial(
        pl.kernel,
        mesh=_SC_MESH,
        out_type=jax.ShapeDtypeStruct((NC, NPAD, D), _F32),
        scratch_types=[
            pltpu.VMEM((cpw, CH), jnp.int32),
            pltpu.VMEM((2, CH, D), _F32),
            pltpu.SemaphoreType.DMA((2,)),
            pltpu.SemaphoreType.DMA((2,)),
            pltpu.VMEM_SHARED((NPAD, D), _F32),
        ],
    )
    def scatter(src3d, z_hbm, zeros_hbm, part_out,
                sidx_v, zrow2, sem_l, sem_a, acc):
        c = lax.axis_index("c")
        s = lax.axis_index("s")
        wid = s * NC + c
        pltpu.sync_copy(zeros_hbm.at[pl.ds(s * RPT, RPT), :],
                        acc.at[pl.ds(s * RPT, RPT), :])
        plsc.subcore_barrier()
        pltpu.sync_copy(src3d.at[wid], sidx_v)

        def l_desc(j, slot):
            o = (wid * cpw + j) * CH
            return pltpu.make_async_copy(z_hbm.at[pl.ds(o, CH), :],
                                         zrow2.at[slot], sem_l.at[slot])

        def a_desc(j, slot):
            return pltpu.make_async_copy(zrow2.at[slot],
                                         acc.at[sidx_v.at[j]],
                                         sem_a.at[slot])

        l_desc(0, 0).start()

        def body(j, carry):
            slot = lax.rem(j, 2)
            nslot = 1 - slot

            @pl.when(j + 1 < cpw)
            def _():
                @pl.when(j >= 1)
                def _():
                    a_desc(j - 1, nslot).wait()
                l_desc(j + 1, nslot).start()

            l_desc(j, slot).wait()
            a_desc(j, slot).start(add=True)
            return carry

        lax.fori_loop(0, cpw, body, 0)
        a_desc(cpw - 2, (cpw - 2) % 2).wait()
        a_desc(cpw - 1, (cpw - 1) % 2).wait()
        plsc.subcore_barrier()
        pltpu.sync_copy(acc.at[pl.ds(s * RPT, RPT), :],
                        part_out.at[c, pl.ds(s * RPT, RPT), :])

    return scatter


_SCATTERS = tuple(_make_scatter(cpw) for cpw in CPWS)


def _combine_body(inp_ref, a_ref, b_ref, c_ref, d_ref, out_ref):
    out_ref[...] = (inp_ref[...] + a_ref[...] + b_ref[...]
                    + c_ref[...] + d_ref[...])


def _run_combine(inp, pa, pb, pc, pd):
    blk = lambda i: (i, 0)
    return pl.pallas_call(
        _combine_body,
        grid=(5,),
        in_specs=[pl.BlockSpec((2000, D), blk)] * 5,
        out_specs=pl.BlockSpec((2000, D), blk),
        out_shape=jax.ShapeDtypeStruct((N, D), _F32),
    )(inp, pa, pb, pc, pd)


def kernel(input, edge_sources, edge_targets, rij, combine_sets, plane_wave,
           cutoff, W_gate, b_gate, g_gate, be_gate, W_mlp, b_mlp, g_mlp,
           be_mlp, W1, b1, W2, b2, W2g, b2g, g2, be2):
    f32 = _F32
    esrc = edge_sources.astype(jnp.int32)
    etgt = edge_targets.astype(jnp.int32)

    def half(x, h):
        return x[HOFF[h]:HOFF[h] + HS[h]]

    src3d = [half(esrc, h).reshape(NW, CPWS[h], CH) for h in range(2)]
    tgt3d = [half(etgt, h).reshape(NW, CPWS[h], CH) for h in range(2)]
    r2d = [half(rij, h).reshape(HS[h], 1) for h in range(2)]
    csh = [half(combine_sets, h) for h in range(2)]
    pwh = [half(plane_wave, h) for h in range(2)]

    pairs = [_GATHERS[h](src3d[h], tgt3d[h], input) for h in range(2)]

    wag = W_gate[:, :D].T
    wbg = W_gate[:, D:2 * D].T
    wcg = W_gate[:, 2 * D:].T
    wam = W_mlp[:, :D].T
    wbm = W_mlp[:, D:2 * D].T
    wcm = W_mlp[:, 2 * D:].T
    w2gt = W2g.T

    stats = [_run_passA(pairs[h][0], pairs[h][1], r2d[h], pwh[h],
                        wag, wbg, wcg, wam, wbm, wcm, w2gt)
             for h in range(2)]
    sg, qg, sm, qm, sy, qy = [a + b for a, b in zip(*stats)]

    eps = 1e-5
    inv_e = 1.0 / E

    def scale_shift(s_, q_, g_, be_):
        mean = s_ * inv_e
        var = q_ * inv_e - mean * mean
        inv = g_.reshape(1, -1) / jnp.sqrt(var + eps)
        return inv, be_.reshape(1, -1) - mean * inv

    scg, shg = scale_shift(sg, qg, g_gate, be_gate)
    scm, shm = scale_shift(sm, qm, g_mlp, be_mlp)
    sc2, sh2 = scale_shift(sy, qy, g2, be2)

    cutf = jnp.full((1, 1), cutoff, f32)
    zeros = jnp.zeros((NPAD, D), f32)
    parts = []
    for h in range(2):
        z = _run_passB(pairs[h][0], pairs[h][1], r2d[h], csh[h], pwh[h],
                       wag, wbg, wcg, wam, wbm, wcm, cutf,
                       scg, shg, scm, shm,
                       W1.T, b1.reshape(1, D), W2.T, b2.reshape(1, D),
                       w2gt, sc2, sh2)
        parts.append(_SCATTERS[h](src3d[h], z, zeros))

    return _run_combine(input, parts[0][0, :N], parts[0][1, :N],
                        parts[1][0, :N], parts[1][1, :N])


# speedup vs baseline: 1.3109x; 1.1265x over previous
import functools

import jax
import jax.numpy as jnp
from jax import lax
from jax.experimental import pallas as pl
from jax.experimental.pallas import tpu as pltpu
from jax.experimental.pallas import tpu_sc as plsc

N = 10000
NPAD = 10240
E = 320000
D = 128
K1 = 16
K2 = 8
NC = 2
NS = 16
NW = NC * NS
CH = 80
BB = 2560
RPT = NPAD // NS
CPWS = (64, 61)
_SC_MESH = plsc.VectorSubcoreMesh(core_axis_name="c", subcore_axis_name="s")
_F32 = jnp.float32
_PREC = None


def _half_sizes():
    sizes = [cpw * CH * NW for cpw in CPWS]
    assert sum(sizes) == E and all(sz % BB == 0 for sz in sizes)
    return sizes


HS = _half_sizes()
HOFF = (0, HS[0])


def _make_gather(cpw):
    eh = cpw * CH * NW

    @functools.partial(
        pl.kernel,
        mesh=_SC_MESH,
        out_type=(
            jax.ShapeDtypeStruct((eh, D), _F32),
            jax.ShapeDtypeStruct((eh, D), _F32),
        ),
        scratch_types=[
            pltpu.VMEM((cpw, CH), jnp.int32),
            pltpu.VMEM((cpw, CH), jnp.int32),
            pltpu.VMEM((2, CH, D), _F32),
            pltpu.VMEM((2, CH, D), _F32),
            pltpu.SemaphoreType.DMA((2,)),
            pltpu.SemaphoreType.DMA((2,)),
        ],
    )
    def gather(src3d, tgt3d, table, ni_out, nj_out,
               sidx_v, tidx_v, srow2, trow2, sem_g, sem_s):
        c = lax.axis_index("c")
        s = lax.axis_index("s")
        wid = s * NC + c
        row0 = wid * cpw
        pltpu.sync_copy(src3d.at[wid], sidx_v)
        pltpu.sync_copy(tgt3d.at[wid], tidx_v)

        def g_desc(j, slot):
            return (pltpu.make_async_copy(table.at[sidx_v.at[j]],
                                          srow2.at[slot], sem_g.at[slot]),
                    pltpu.make_async_copy(table.at[tidx_v.at[j]],
                                          trow2.at[slot], sem_g.at[slot]))

        def s_desc(j, slot):
            o = (row0 + j) * CH
            return (pltpu.make_async_copy(srow2.at[slot],
                                          ni_out.at[pl.ds(o, CH), :],
                                          sem_s.at[slot]),
                    pltpu.make_async_copy(trow2.at[slot],
                                          nj_out.at[pl.ds(o, CH), :],
                                          sem_s.at[slot]))

        for d in g_desc(0, 0):
            d.start()

        def body(j, carry):
            slot = lax.rem(j, 2)
            nslot = 1 - slot

            @pl.when(j + 1 < cpw)
            def _():
                @pl.when(j >= 1)
                def _():
                    for d in s_desc(j - 1, nslot):
                        d.wait()
                for d in g_desc(j + 1, nslot):
                    d.start()

            for d in g_desc(j, slot):
                d.wait()
            for d in s_desc(j, slot):
                d.start()
            return carry

        lax.fori_loop(0, cpw, body, 0)
        for d in s_desc(cpw - 2, (cpw - 2) % 2):
            d.wait()
        for d in s_desc(cpw - 1, (cpw - 1) % 2):
            d.wait()

    return gather


_GATHERS = tuple(_make_gather(cpw) for cpw in CPWS)


def _compute_x(ni, nj, r, wag, wbg, wcg, wam, wbm, wcm):
    delta = (ni - nj) / r
    xg = (jnp.dot(ni, wag, precision=_PREC)
          + jnp.dot(nj, wbg, precision=_PREC)
          + jnp.dot(delta, wcg, precision=_PREC))
    xm = (jnp.dot(ni, wam, precision=_PREC)
          + jnp.dot(nj, wbm, precision=_PREC)
          + jnp.dot(delta, wcm, precision=_PREC))
    return xg, xm


def _passA_body(ni_ref, nj_ref, r_ref, pw_ref,
                wag, wbg, wcg, wam, wbm, wcm, w2gt,
                sg, qg, sm, qm, sy, qy):
    i = pl.program_id(0)
    xg, xm = _compute_x(ni_ref[...], nj_ref[...], r_ref[...],
                        wag[...], wbg[...], wcg[...],
                        wam[...], wbm[...], wcm[...])
    y = jnp.dot(pw_ref[...], w2gt[...], precision=_PREC)
    bs_g = jnp.sum(xg, axis=0, keepdims=True)
    bq_g = jnp.sum(xg * xg, axis=0, keepdims=True)
    bs_m = jnp.sum(xm, axis=0, keepdims=True)
    bq_m = jnp.sum(xm * xm, axis=0, keepdims=True)
    bs_y = jnp.sum(y, axis=0, keepdims=True)
    bq_y = jnp.sum(y * y, axis=0, keepdims=True)

    @pl.when(i == 0)
    def _():
        sg[...] = bs_g
        qg[...] = bq_g
        sm[...] = bs_m
        qm[...] = bq_m
        sy[...] = bs_y
        qy[...] = bq_y

    @pl.when(i != 0)
    def _():
        sg[...] += bs_g
        qg[...] += bq_g
        sm[...] += bs_m
        qm[...] += bq_m
        sy[...] += bs_y
        qy[...] += bq_y


def _run_passA(ni, nj, r2d, pw, wag, wbg, wcg, wam, wbm, wcm, w2gt):
    nblk = ni.shape[0] // BB
    blk = lambda i: (i, 0)
    cst = lambda i: (0, 0)
    return pl.pallas_call(
        _passA_body,
        grid=(nblk,),
        in_specs=[
            pl.BlockSpec((BB, D), blk),
            pl.BlockSpec((BB, D), blk),
            pl.BlockSpec((BB, 1), blk),
            pl.BlockSpec((BB, K2), blk),
            pl.BlockSpec((D, D), cst),
            pl.BlockSpec((D, D), cst),
            pl.BlockSpec((D, D), cst),
            pl.BlockSpec((D, D), cst),
            pl.BlockSpec((D, D), cst),
            pl.BlockSpec((D, D), cst),
            pl.BlockSpec((K2, K2), cst),
        ],
        out_specs=[
            pl.BlockSpec((1, D), cst),
            pl.BlockSpec((1, D), cst),
            pl.BlockSpec((1, D), cst),
            pl.BlockSpec((1, D), cst),
            pl.BlockSpec((1, K2), cst),
            pl.BlockSpec((1, K2), cst),
        ],
        out_shape=[
            jax.ShapeDtypeStruct((1, D), _F32),
            jax.ShapeDtypeStruct((1, D), _F32),
            jax.ShapeDtypeStruct((1, D), _F32),
            jax.ShapeDtypeStruct((1, D), _F32),
            jax.ShapeDtypeStruct((1, K2), _F32),
            jax.ShapeDtypeStruct((1, K2), _F32),
        ],
    )(ni, nj, r2d, pw, wag, wbg, wcg, wam, wbm, wcm, w2gt)


def _passB_body(ni_ref, nj_ref, r_ref, cs_ref, pw_ref,
                wag, wbg, wcg, wam, wbm, wcm, cutf,
                scg, shg, scm, shm, w1t, b1, w2t, b2, w2gt, sc2, sh2,
                z_ref):
    xg, xm = _compute_x(ni_ref[...], nj_ref[...], r_ref[...],
                        wag[...], wbg[...], wcg[...],
                        wam[...], wbm[...], wcm[...])
    eg = xg * scg[...] + shg[...]
    em = xm * scm[...] + shm[...]
    z1 = jnp.dot(cs_ref[...], w1t[...], precision=_PREC) + b1[...]
    pw = pw_ref[...]
    y = jnp.dot(pw, w2gt[...], precision=_PREC)
    gate = y * sc2[...] + sh2[...]
    z2 = jnp.dot(pw * gate, w2t[...], precision=_PREC) + b2[...]
    mask = (r_ref[...] < cutf[...]).astype(_F32)
    z_ref[...] = eg * em * (z1 + z2) * mask


def _run_passB(ni, nj, r2d, cs, pw, wag, wbg, wcg, wam, wbm, wcm,
               cutf, scg, shg, scm, shm, w1t, b1, w2t, b2, w2gt, sc2, sh2):
    nblk = ni.shape[0] // BB
    blk = lambda i: (i, 0)
    cst = lambda i: (0, 0)
    return pl.pallas_call(
        _passB_body,
        grid=(nblk,),
        in_specs=[
            pl.BlockSpec((BB, D), blk),
            pl.BlockSpec((BB, D), blk),
            pl.BlockSpec((BB, 1), blk),
            pl.BlockSpec((BB, K1), blk),
            pl.BlockSpec((BB, K2), blk),
            pl.BlockSpec((D, D), cst),
            pl.BlockSpec((D, D), cst),
            pl.BlockSpec((D, D), cst),
            pl.BlockSpec((D, D), cst),
            pl.BlockSpec((D, D), cst),
            pl.BlockSpec((D, D), cst),
            pl.BlockSpec((1, 1), cst),
            pl.BlockSpec((1, D), cst),
            pl.BlockSpec((1, D), cst),
            pl.BlockSpec((1, D), cst),
            pl.BlockSpec((1, D), cst),
            pl.BlockSpec((K1, D), cst),
            pl.BlockSpec((1, D), cst),
            pl.BlockSpec((K2, D), cst),
            pl.BlockSpec((1, D), cst),
            pl.BlockSpec((K2, K2), cst),
            pl.BlockSpec((1, K2), cst),
            pl.BlockSpec((1, K2), cst),
        ],
        out_specs=pl.BlockSpec((BB, D), blk),
        out_shape=jax.ShapeDtypeStruct((ni.shape[0], D), _F32),
    )(ni, nj, r2d, cs, pw, wag, wbg, wcg, wam, wbm, wcm,
      cutf, scg, shg, scm, shm, w1t, b1, w2t, b2, w2gt, sc2, sh2)


def _make_scatter(cpw):
    eh = cpw * CH * NW

    @functools.partial(
        pl.kernel,
        mesh=_SC_MESH,
        out_type=jax.ShapeDtypeStruct((NC, NPAD, D), _F32),
        scratch_types=[
            pltpu.VMEM((cpw, CH), jnp.int32),
            pltpu.VMEM((2, CH, D), _F32),
            pltpu.SemaphoreType.DMA((2,)),
            pltpu.SemaphoreType.DMA((2,)),
            pltpu.VMEM_SHARED((NPAD, D), _F32),
        ],
    )
    def scatter(src3d, z_hbm, zeros_hbm, part_out,
                sidx_v, zrow2, sem_l, sem_a, acc):
        c = lax.axis_index("c")
        s = lax.axis_index("s")
        wid = s * NC + c
        pltpu.sync_copy(zeros_hbm.at[pl.ds(s * RPT, RPT), :],
                        acc.at[pl.ds(s * RPT, RPT), :])
        plsc.subcore_barrier()
        pltpu.sync_copy(src3d.at[wid], sidx_v)

        def l_desc(j, slot):
            o = (wid * cpw + j) * CH
            return pltpu.make_async_copy(z_hbm.at[pl.ds(o, CH), :],
                                         zrow2.at[slot], sem_l.at[slot])

        def a_desc(j, slot):
            return pltpu.make_async_copy(zrow2.at[slot],
                                         acc.at[sidx_v.at[j]],
                                         sem_a.at[slot])

        l_desc(0, 0).start()

        def body(j, carry):
            slot = lax.rem(j, 2)
            nslot = 1 - slot

            @pl.when(j + 1 < cpw)
            def _():
                @pl.when(j >= 1)
                def _():
                    a_desc(j - 1, nslot).wait()
                l_desc(j + 1, nslot).start()

            l_desc(j, slot).wait()
            a_desc(j, slot).start(add=True)
            return carry

        lax.fori_loop(0, cpw, body, 0)
        a_desc(cpw - 2, (cpw - 2) % 2).wait()
        a_desc(cpw - 1, (cpw - 1) % 2).wait()
        plsc.subcore_barrier()
        pltpu.sync_copy(acc.at[pl.ds(s * RPT, RPT), :],
                        part_out.at[c, pl.ds(s * RPT, RPT), :])

    return scatter


_SCATTERS = tuple(_make_scatter(cpw) for cpw in CPWS)


def _combine_body(inp_ref, a_ref, b_ref, c_ref, d_ref, out_ref):
    out_ref[...] = (inp_ref[...] + a_ref[...] + b_ref[...]
                    + c_ref[...] + d_ref[...])


def _run_combine(inp, pa, pb, pc, pd):
    blk = lambda i: (i, 0)
    return pl.pallas_call(
        _combine_body,
        grid=(5,),
        in_specs=[pl.BlockSpec((2000, D), blk)] * 5,
        out_specs=pl.BlockSpec((2000, D), blk),
        out_shape=jax.ShapeDtypeStruct((N, D), _F32),
    )(inp, pa, pb, pc, pd)


def kernel(input, edge_sources, edge_targets, rij, combine_sets, plane_wave,
           cutoff, W_gate, b_gate, g_gate, be_gate, W_mlp, b_mlp, g_mlp,
           be_mlp, W1, b1, W2, b2, W2g, b2g, g2, be2):
    f32 = _F32
    esrc = edge_sources.astype(jnp.int32)
    etgt = edge_targets.astype(jnp.int32)

    def half(x, h):
        return x[HOFF[h]:HOFF[h] + HS[h]]

    src3d = [half(esrc, h).reshape(NW, CPWS[h], CH) for h in range(2)]
    tgt3d = [half(etgt, h).reshape(NW, CPWS[h], CH) for h in range(2)]
    r2d = [half(rij, h).reshape(HS[h], 1) for h in range(2)]
    csh = [half(combine_sets, h) for h in range(2)]
    pwh = [half(plane_wave, h) for h in range(2)]

    pairs = [_GATHERS[h](src3d[h], tgt3d[h], input) for h in range(2)]

    wag = W_gate[:, :D].T
    wbg = W_gate[:, D:2 * D].T
    wcg = W_gate[:, 2 * D:].T
    wam = W_mlp[:, :D].T
    wbm = W_mlp[:, D:2 * D].T
    wcm = W_mlp[:, 2 * D:].T
    w2gt = W2g.T

    stats = [_run_passA(pairs[h][0], pairs[h][1], r2d[h], pwh[h],
                        wag, wbg, wcg, wam, wbm, wcm, w2gt)
             for h in range(2)]
    sg, qg, sm, qm, sy, qy = [a + b for a, b in zip(*stats)]

    eps = 1e-5
    inv_e = 1.0 / E

    def scale_shift(s_, q_, g_, be_):
        mean = s_ * inv_e
        var = q_ * inv_e - mean * mean
        inv = g_.reshape(1, -1) / jnp.sqrt(var + eps)
        return inv, be_.reshape(1, -1) - mean * inv

    scg, shg = scale_shift(sg, qg, g_gate, be_gate)
    scm, shm = scale_shift(sm, qm, g_mlp, be_mlp)
    sc2, sh2 = scale_shift(sy, qy, g2, be2)

    cutf = jnp.full((1, 1), cutoff, f32)
    zeros = jnp.zeros((NPAD, D), f32)
    parts = []
    for h in range(2):
        z = _run_passB(pairs[h][0], pairs[h][1], r2d[h], csh[h], pwh[h],
                       wag, wbg, wcg, wam, wbm, wcm, cutf,
                       scg, shg, scm, shm,
                       W1.T, b1.reshape(1, D), W2.T, b2.reshape(1, D),
                       w2gt, sc2, sh2)
        parts.append(_SCATTERS[h](src3d[h], z, zeros))

    return _run_combine(input, parts[0][0, :N], parts[0][1, :N],
                        parts[1][0, :N], parts[1][1, :N])


# speedup vs baseline: 1.4002x; 1.0681x over previous
import functools

import jax
import jax.numpy as jnp
from jax import lax
from jax.experimental import pallas as pl
from jax.experimental.pallas import tpu as pltpu
from jax.experimental.pallas import tpu_sc as plsc

N = 10000
NPAD = 10240
E = 320000
D = 128
K1 = 16
K2 = 8
NC = 2
NS = 16
NW = NC * NS
CH = 80
BB = 6400
RPT = NPAD // NS
CPWS = (125,)
NH = len(CPWS)
_SC_MESH = plsc.VectorSubcoreMesh(core_axis_name="c", subcore_axis_name="s")
_F32 = jnp.float32
_PREC = None


def _half_sizes():
    sizes = [cpw * CH * NW for cpw in CPWS]
    assert sum(sizes) == E and all(sz % BB == 0 for sz in sizes)
    return sizes


HS = _half_sizes()
HOFF = tuple(sum(HS[:h]) for h in range(NH))


def _make_gather(cpw):
    eh = cpw * CH * NW

    @functools.partial(
        pl.kernel,
        mesh=_SC_MESH,
        out_type=(
            jax.ShapeDtypeStruct((eh, D), _F32),
            jax.ShapeDtypeStruct((eh, D), _F32),
        ),
        scratch_types=[
            pltpu.VMEM((cpw, CH), jnp.int32),
            pltpu.VMEM((cpw, CH), jnp.int32),
            pltpu.VMEM((2, CH, D), _F32),
            pltpu.VMEM((2, CH, D), _F32),
            pltpu.SemaphoreType.DMA((2,)),
            pltpu.SemaphoreType.DMA((2,)),
        ],
    )
    def gather(src3d, tgt3d, table, ni_out, nj_out,
               sidx_v, tidx_v, srow2, trow2, sem_g, sem_s):
        c = lax.axis_index("c")
        s = lax.axis_index("s")
        wid = s * NC + c
        row0 = wid * cpw
        pltpu.sync_copy(src3d.at[wid], sidx_v)
        pltpu.sync_copy(tgt3d.at[wid], tidx_v)

        def g_desc(j, slot):
            return (pltpu.make_async_copy(table.at[sidx_v.at[j]],
                                          srow2.at[slot], sem_g.at[slot]),
                    pltpu.make_async_copy(table.at[tidx_v.at[j]],
                                          trow2.at[slot], sem_g.at[slot]))

        def s_desc(j, slot):
            o = (row0 + j) * CH
            return (pltpu.make_async_copy(srow2.at[slot],
                                          ni_out.at[pl.ds(o, CH), :],
                                          sem_s.at[slot]),
                    pltpu.make_async_copy(trow2.at[slot],
                                          nj_out.at[pl.ds(o, CH), :],
                                          sem_s.at[slot]))

        for d in g_desc(0, 0):
            d.start()

        def body(j, carry):
            slot = lax.rem(j, 2)
            nslot = 1 - slot

            @pl.when(j + 1 < cpw)
            def _():
                @pl.when(j >= 1)
                def _():
                    for d in s_desc(j - 1, nslot):
                        d.wait()
                for d in g_desc(j + 1, nslot):
                    d.start()

            for d in g_desc(j, slot):
                d.wait()
            for d in s_desc(j, slot):
                d.start()
            return carry

        lax.fori_loop(0, cpw, body, 0)
        for d in s_desc(cpw - 2, (cpw - 2) % 2):
            d.wait()
        for d in s_desc(cpw - 1, (cpw - 1) % 2):
            d.wait()

    return gather


_GATHERS = tuple(_make_gather(cpw) for cpw in CPWS)


def _compute_x(ni, nj, r, wag, wbg, wcg, wam, wbm, wcm):
    delta = (ni - nj) / r
    xg = (jnp.dot(ni, wag, precision=_PREC)
          + jnp.dot(nj, wbg, precision=_PREC)
          + jnp.dot(delta, wcg, precision=_PREC))
    xm = (jnp.dot(ni, wam, precision=_PREC)
          + jnp.dot(nj, wbm, precision=_PREC)
          + jnp.dot(delta, wcm, precision=_PREC))
    return xg, xm


def _passA_body(ni_ref, nj_ref, r_ref, pw_ref,
                wag, wbg, wcg, wam, wbm, wcm, w2gt,
                sg, qg, sm, qm, sy, qy):
    i = pl.program_id(0)
    xg, xm = _compute_x(ni_ref[...], nj_ref[...], r_ref[...],
                        wag[...], wbg[...], wcg[...],
                        wam[...], wbm[...], wcm[...])
    y = jnp.dot(pw_ref[...], w2gt[...], precision=_PREC)
    bs_g = jnp.sum(xg, axis=0, keepdims=True)
    bq_g = jnp.sum(xg * xg, axis=0, keepdims=True)
    bs_m = jnp.sum(xm, axis=0, keepdims=True)
    bq_m = jnp.sum(xm * xm, axis=0, keepdims=True)
    bs_y = jnp.sum(y, axis=0, keepdims=True)
    bq_y = jnp.sum(y * y, axis=0, keepdims=True)

    @pl.when(i == 0)
    def _():
        sg[...] = bs_g
        qg[...] = bq_g
        sm[...] = bs_m
        qm[...] = bq_m
        sy[...] = bs_y
        qy[...] = bq_y

    @pl.when(i != 0)
    def _():
        sg[...] += bs_g
        qg[...] += bq_g
        sm[...] += bs_m
        qm[...] += bq_m
        sy[...] += bs_y
        qy[...] += bq_y


def _run_passA(ni, nj, r2d, pw, wag, wbg, wcg, wam, wbm, wcm, w2gt):
    nblk = ni.shape[0] // BB
    blk = lambda i: (i, 0)
    cst = lambda i: (0, 0)
    return pl.pallas_call(
        _passA_body,
        grid=(nblk,),
        in_specs=[
            pl.BlockSpec((BB, D), blk),
            pl.BlockSpec((BB, D), blk),
            pl.BlockSpec((BB, 1), blk),
            pl.BlockSpec((BB, K2), blk),
            pl.BlockSpec((D, D), cst),
            pl.BlockSpec((D, D), cst),
            pl.BlockSpec((D, D), cst),
            pl.BlockSpec((D, D), cst),
            pl.BlockSpec((D, D), cst),
            pl.BlockSpec((D, D), cst),
            pl.BlockSpec((K2, K2), cst),
        ],
        out_specs=[
            pl.BlockSpec((1, D), cst),
            pl.BlockSpec((1, D), cst),
            pl.BlockSpec((1, D), cst),
            pl.BlockSpec((1, D), cst),
            pl.BlockSpec((1, K2), cst),
            pl.BlockSpec((1, K2), cst),
        ],
        out_shape=[
            jax.ShapeDtypeStruct((1, D), _F32),
            jax.ShapeDtypeStruct((1, D), _F32),
            jax.ShapeDtypeStruct((1, D), _F32),
            jax.ShapeDtypeStruct((1, D), _F32),
            jax.ShapeDtypeStruct((1, K2), _F32),
            jax.ShapeDtypeStruct((1, K2), _F32),
        ],
    )(ni, nj, r2d, pw, wag, wbg, wcg, wam, wbm, wcm, w2gt)


def _passB_body(ni_ref, nj_ref, r_ref, cs_ref, pw_ref,
                wag, wbg, wcg, wam, wbm, wcm, cutf,
                scg, shg, scm, shm, w1t, b1, w2t, b2, w2gt, sc2, sh2,
                z_ref):
    xg, xm = _compute_x(ni_ref[...], nj_ref[...], r_ref[...],
                        wag[...], wbg[...], wcg[...],
                        wam[...], wbm[...], wcm[...])
    eg = xg * scg[...] + shg[...]
    em = xm * scm[...] + shm[...]
    z1 = jnp.dot(cs_ref[...], w1t[...], precision=_PREC) + b1[...]
    pw = pw_ref[...]
    y = jnp.dot(pw, w2gt[...], precision=_PREC)
    gate = y * sc2[...] + sh2[...]
    z2 = jnp.dot(pw * gate, w2t[...], precision=_PREC) + b2[...]
    mask = (r_ref[...] < cutf[...]).astype(_F32)
    z_ref[...] = eg * em * (z1 + z2) * mask


def _run_passB(ni, nj, r2d, cs, pw, wag, wbg, wcg, wam, wbm, wcm,
               cutf, scg, shg, scm, shm, w1t, b1, w2t, b2, w2gt, sc2, sh2):
    nblk = ni.shape[0] // BB
    blk = lambda i: (i, 0)
    cst = lambda i: (0, 0)
    return pl.pallas_call(
        _passB_body,
        grid=(nblk,),
        in_specs=[
            pl.BlockSpec((BB, D), blk),
            pl.BlockSpec((BB, D), blk),
            pl.BlockSpec((BB, 1), blk),
            pl.BlockSpec((BB, K1), blk),
            pl.BlockSpec((BB, K2), blk),
            pl.BlockSpec((D, D), cst),
            pl.BlockSpec((D, D), cst),
            pl.BlockSpec((D, D), cst),
            pl.BlockSpec((D, D), cst),
            pl.BlockSpec((D, D), cst),
            pl.BlockSpec((D, D), cst),
            pl.BlockSpec((1, 1), cst),
            pl.BlockSpec((1, D), cst),
            pl.BlockSpec((1, D), cst),
            pl.BlockSpec((1, D), cst),
            pl.BlockSpec((1, D), cst),
            pl.BlockSpec((K1, D), cst),
            pl.BlockSpec((1, D), cst),
            pl.BlockSpec((K2, D), cst),
            pl.BlockSpec((1, D), cst),
            pl.BlockSpec((K2, K2), cst),
            pl.BlockSpec((1, K2), cst),
            pl.BlockSpec((1, K2), cst),
        ],
        out_specs=pl.BlockSpec((BB, D), blk),
        out_shape=jax.ShapeDtypeStruct((ni.shape[0], D), _F32),
    )(ni, nj, r2d, cs, pw, wag, wbg, wcg, wam, wbm, wcm,
      cutf, scg, shg, scm, shm, w1t, b1, w2t, b2, w2gt, sc2, sh2)


def _make_scatter(cpw):
    eh = cpw * CH * NW

    @functools.partial(
        pl.kernel,
        mesh=_SC_MESH,
        out_type=jax.ShapeDtypeStruct((NC, NPAD, D), _F32),
        scratch_types=[
            pltpu.VMEM((cpw, CH), jnp.int32),
            pltpu.VMEM((2, CH, D), _F32),
            pltpu.SemaphoreType.DMA((2,)),
            pltpu.SemaphoreType.DMA((2,)),
            pltpu.VMEM_SHARED((NPAD, D), _F32),
        ],
    )
    def scatter(src3d, z_hbm, zeros_hbm, part_out,
                sidx_v, zrow2, sem_l, sem_a, acc):
        c = lax.axis_index("c")
        s = lax.axis_index("s")
        wid = s * NC + c
        pltpu.sync_copy(zeros_hbm.at[pl.ds(s * RPT, RPT), :],
                        acc.at[pl.ds(s * RPT, RPT), :])
        plsc.subcore_barrier()
        pltpu.sync_copy(src3d.at[wid], sidx_v)

        def l_desc(j, slot):
            o = (wid * cpw + j) * CH
            return pltpu.make_async_copy(z_hbm.at[pl.ds(o, CH), :],
                                         zrow2.at[slot], sem_l.at[slot])

        def a_desc(j, slot):
            return pltpu.make_async_copy(zrow2.at[slot],
                                         acc.at[sidx_v.at[j]],
                                         sem_a.at[slot])

        l_desc(0, 0).start()

        def body(j, carry):
            slot = lax.rem(j, 2)
            nslot = 1 - slot

            @pl.when(j + 1 < cpw)
            def _():
                @pl.when(j >= 1)
                def _():
                    a_desc(j - 1, nslot).wait()
                l_desc(j + 1, nslot).start()

            l_desc(j, slot).wait()
            a_desc(j, slot).start(add=True)
            return carry

        lax.fori_loop(0, cpw, body, 0)
        a_desc(cpw - 2, (cpw - 2) % 2).wait()
        a_desc(cpw - 1, (cpw - 1) % 2).wait()
        plsc.subcore_barrier()
        pltpu.sync_copy(acc.at[pl.ds(s * RPT, RPT), :],
                        part_out.at[c, pl.ds(s * RPT, RPT), :])

    return scatter


_SCATTERS = tuple(_make_scatter(cpw) for cpw in CPWS)


def _combine_body(*refs):
    out_ref = refs[-1]
    acc = refs[0][...]
    for r in refs[1:-1]:
        acc = acc + r[...]
    out_ref[...] = acc


def _run_combine(inp, *parts):
    blk = lambda i: (i, 0)
    n_in = 1 + len(parts)
    return pl.pallas_call(
        _combine_body,
        grid=(5,),
        in_specs=[pl.BlockSpec((2000, D), blk)] * n_in,
        out_specs=pl.BlockSpec((2000, D), blk),
        out_shape=jax.ShapeDtypeStruct((N, D), _F32),
    )(inp, *parts)


def kernel(input, edge_sources, edge_targets, rij, combine_sets, plane_wave,
           cutoff, W_gate, b_gate, g_gate, be_gate, W_mlp, b_mlp, g_mlp,
           be_mlp, W1, b1, W2, b2, W2g, b2g, g2, be2):
    f32 = _F32
    esrc = edge_sources.astype(jnp.int32)
    etgt = edge_targets.astype(jnp.int32)

    def half(x, h):
        return x[HOFF[h]:HOFF[h] + HS[h]]

    src3d = [half(esrc, h).reshape(NW, CPWS[h], CH) for h in range(NH)]
    tgt3d = [half(etgt, h).reshape(NW, CPWS[h], CH) for h in range(NH)]
    r2d = [half(rij, h).reshape(HS[h], 1) for h in range(NH)]
    csh = [half(combine_sets, h) for h in range(NH)]
    pwh = [half(plane_wave, h) for h in range(NH)]

    pairs = [_GATHERS[h](src3d[h], tgt3d[h], input) for h in range(NH)]

    wag = W_gate[:, :D].T
    wbg = W_gate[:, D:2 * D].T
    wcg = W_gate[:, 2 * D:].T
    wam = W_mlp[:, :D].T
    wbm = W_mlp[:, D:2 * D].T
    wcm = W_mlp[:, 2 * D:].T
    w2gt = W2g.T

    stats = [_run_passA(pairs[h][0], pairs[h][1], r2d[h], pwh[h],
                        wag, wbg, wcg, wam, wbm, wcm, w2gt)
             for h in range(NH)]
    sg, qg, sm, qm, sy, qy = [sum(xs[1:], xs[0]) for xs in zip(*stats)]

    eps = 1e-5
    inv_e = 1.0 / E

    def scale_shift(s_, q_, g_, be_):
        mean = s_ * inv_e
        var = q_ * inv_e - mean * mean
        inv = g_.reshape(1, -1) / jnp.sqrt(var + eps)
        return inv, be_.reshape(1, -1) - mean * inv

    scg, shg = scale_shift(sg, qg, g_gate, be_gate)
    scm, shm = scale_shift(sm, qm, g_mlp, be_mlp)
    sc2, sh2 = scale_shift(sy, qy, g2, be2)

    cutf = jnp.full((1, 1), cutoff, f32)
    zeros = jnp.zeros((NPAD, D), f32)
    parts = []
    for h in range(NH):
        z = _run_passB(pairs[h][0], pairs[h][1], r2d[h], csh[h], pwh[h],
                       wag, wbg, wcg, wam, wbm, wcm, cutf,
                       scg, shg, scm, shm,
                       W1.T, b1.reshape(1, D), W2.T, b2.reshape(1, D),
                       w2gt, sc2, sh2)
        parts.append(_SCATTERS[h](src3d[h], z, zeros))

    flat = [pt[c, :N] for pt in parts for c in range(NC)]
    return _run_combine(input, *flat)
